# Initial kernel scaffold; baseline (speedup 1.0000x reference)
import jax
import jax.numpy as jnp
from jax.experimental import pallas as pl


def kernel(x, table, W, b):
    raise NotImplementedError("write your pallas kernel here")


# speedup vs baseline: 5.2282x; 5.2282x over previous
import functools

import numpy as np
import jax
import jax.numpy as jnp
from jax import lax
from jax.experimental import pallas as pl
from jax.experimental.pallas import tpu as pltpu
from jax.experimental.pallas import tpu_sc as plsc

_VOCAB, _EMBED, _OUT = 100000, 32, 128
_B, _L = 4096, 50
_NW = 32
_CH = 128
_RPW = _B // _NW
_IPW = _RPW * _L
_NCH = _IPW // _CH

_DST3 = (
    (np.arange(_IPW, dtype=np.int32) // _L).reshape(1, _NCH, _CH)
    + (np.arange(16, dtype=np.int32) * _RPW).reshape(16, 1, 1)
)

_mesh = plsc.VectorSubcoreMesh(core_axis_name="c", subcore_axis_name="s")


@functools.partial(
    pl.kernel,
    mesh=_mesh,
    compiler_params=pltpu.CompilerParams(use_tc_tiling_on_sc=False),
    out_type=jax.ShapeDtypeStruct((_B, _EMBED), jnp.float32),
    scratch_types=[
        pltpu.VMEM((_CH,), jnp.int32),
        pltpu.VMEM((_CH,), jnp.int32),
        pltpu.VMEM((_CH, _EMBED), jnp.float32),
        pltpu.VMEM((_RPW, _EMBED), jnp.float32),
        pltpu.VMEM_SHARED((16 * _RPW, _EMBED), jnp.float32),
        pltpu.SemaphoreType.DMA,
    ],
)
def _sc_pool(x2, table, dst3, pooled, idx_v, dst_v, rows_v, zero_v, pooled_sh,
             sem):
    cid = lax.axis_index("c")
    sid = lax.axis_index("s")
    wid = sid * 2 + cid

    z = jnp.zeros((16,), jnp.float32)

    def zero_row(r, carry):
        zero_v[r, pl.ds(0, 16)] = z
        zero_v[r, pl.ds(16, 16)] = z
        return carry

    lax.fori_loop(0, _RPW, zero_row, 0)
    pltpu.sync_copy(zero_v, pooled_sh.at[pl.ds(sid * _RPW, _RPW)])

    def chunk(c, carry):
        pltpu.sync_copy(x2.at[wid * _NCH + c], idx_v)
        pltpu.sync_copy(dst3.at[sid, c], dst_v)
        pltpu.async_copy(table.at[idx_v], rows_v, sem).wait()
        pltpu.sync_copy(rows_v, pooled_sh.at[dst_v], add=True)
        return carry

    lax.fori_loop(0, _NCH, chunk, 0)
    pltpu.sync_copy(
        pooled_sh.at[pl.ds(sid * _RPW, _RPW)],
        pooled.at[pl.ds(wid * _RPW, _RPW)],
    )


def _lin_body(p_ref, w_ref, b_ref, o_ref):
    o_ref[...] = (
        jnp.dot(p_ref[...], w_ref[...], preferred_element_type=jnp.float32)
        + b_ref[...]
    )


def _linear(pooled, wt, b2):
    blk = 512
    return pl.pallas_call(
        _lin_body,
        grid=(_B // blk,),
        in_specs=[
            pl.BlockSpec((blk, _EMBED), lambda i: (i, 0)),
            pl.BlockSpec((_EMBED, _OUT), lambda i: (0, 0)),
            pl.BlockSpec((1, _OUT), lambda i: (0, 0)),
        ],
        out_specs=pl.BlockSpec((blk, _OUT), lambda i: (i, 0)),
        out_shape=jax.ShapeDtypeStruct((_B, _OUT), jnp.float32),
    )(pooled, wt, b2)


def kernel(x, table, W, b):
    x2 = x.reshape(_NW * _NCH, _CH)
    pooled = _sc_pool(x2, table, _DST3)
    return _linear(pooled, W.T, b.reshape(1, _OUT))


# speedup vs baseline: 8.5439x; 1.6342x over previous
import functools

import numpy as np
import jax
import jax.numpy as jnp
from jax import lax
from jax.experimental import pallas as pl
from jax.experimental.pallas import tpu as pltpu
from jax.experimental.pallas import tpu_sc as plsc

_VOCAB, _EMBED, _OUT = 100000, 32, 128
_B, _L = 4096, 50
_NW = 32
_CH = 128
_RPW = _B // _NW
_IPW = _RPW * _L
_NCH = _IPW // _CH

_DST3 = (
    (np.arange(_IPW, dtype=np.int32) // _L).reshape(1, _NCH, _CH)
    + (np.arange(16, dtype=np.int32) * _RPW).reshape(16, 1, 1)
)

_mesh = plsc.VectorSubcoreMesh(core_axis_name="c", subcore_axis_name="s")


@functools.partial(
    pl.kernel,
    mesh=_mesh,
    compiler_params=pltpu.CompilerParams(use_tc_tiling_on_sc=False),
    out_type=jax.ShapeDtypeStruct((_B, _EMBED), jnp.float32),
    scratch_types=[
        pltpu.VMEM((_NCH, _CH), jnp.int32),
        pltpu.VMEM((_NCH, _CH), jnp.int32),
        pltpu.VMEM((_CH, _EMBED), jnp.float32),
        pltpu.VMEM((_CH, _EMBED), jnp.float32),
        pltpu.VMEM((_RPW, _EMBED), jnp.float32),
        pltpu.VMEM_SHARED((16 * _RPW, _EMBED), jnp.float32),
        pltpu.SemaphoreType.DMA,
        pltpu.SemaphoreType.DMA,
    ],
)
def _sc_pool(x2, table, dst3, pooled, idx_all, dst_all, rows0, rows1, zero_v,
             pooled_sh, gsem0, gsem1):
    cid = lax.axis_index("c")
    sid = lax.axis_index("s")
    wid = sid * 2 + cid

    pltpu.sync_copy(x2.at[pl.ds(wid * _NCH, _NCH)], idx_all)
    pltpu.sync_copy(dst3.at[sid], dst_all)

    z = jnp.zeros((16,), jnp.float32)

    def zero_row(r, carry):
        zero_v[r, pl.ds(0, 16)] = z
        zero_v[r, pl.ds(16, 16)] = z
        return carry

    lax.fori_loop(0, _RPW, zero_row, 0)
    pltpu.sync_copy(zero_v, pooled_sh.at[pl.ds(sid * _RPW, _RPW)])

    pltpu.async_copy(table.at[idx_all.at[0]], rows0, gsem0)

    def outer(cc, carry):
        c = cc * 2
        pltpu.async_copy(table.at[idx_all.at[c + 1]], rows1, gsem1)
        pltpu.make_async_copy(table.at[idx_all.at[c]], rows0, gsem0).wait()
        pltpu.sync_copy(rows0, pooled_sh.at[dst_all.at[c]], add=True)

        @pl.when(cc < _NCH // 2 - 1)
        def _():
            pltpu.async_copy(table.at[idx_all.at[c + 2]], rows0, gsem0)

        pltpu.make_async_copy(table.at[idx_all.at[c + 1]], rows1, gsem1).wait()
        pltpu.sync_copy(rows1, pooled_sh.at[dst_all.at[c + 1]], add=True)
        return carry

    lax.fori_loop(0, _NCH // 2, outer, 0)

    pltpu.sync_copy(
        pooled_sh.at[pl.ds(sid * _RPW, _RPW)],
        pooled.at[pl.ds(wid * _RPW, _RPW)],
    )


def _lin_body(p_ref, w_ref, b_ref, o_ref):
    o_ref[...] = (
        jnp.dot(p_ref[...], w_ref[...], preferred_element_type=jnp.float32)
        + b_ref[...]
    )


def _linear(pooled, wt, b2):
    blk = 512
    return pl.pallas_call(
        _lin_body,
        grid=(_B // blk,),
        in_specs=[
            pl.BlockSpec((blk, _EMBED), lambda i: (i, 0)),
            pl.BlockSpec((_EMBED, _OUT), lambda i: (0, 0)),
            pl.BlockSpec((1, _OUT), lambda i: (0, 0)),
        ],
        out_specs=pl.BlockSpec((blk, _OUT), lambda i: (i, 0)),
        out_shape=jax.ShapeDtypeStruct((_B, _OUT), jnp.float32),
    )(pooled, wt, b2)


def kernel(x, table, W, b):
    x2 = x.reshape(_NW * _NCH, _CH)
    pooled = _sc_pool(x2, table, _DST3)
    return _linear(pooled, W.T, b.reshape(1, _OUT))
